# baseline (speedup 1.0000x reference)
import functools
import math

import jax
import jax.numpy as jnp
from jax import lax
from jax.experimental import pallas as pl
from jax.experimental.pallas import tpu as pltpu
from jax.experimental.pallas import tpu_sc as plsc

_D = 128
_L = 16
_NC = 2
_NS = 16
_NW = _NC * _NS
_C = 128
_NBUF = 4
_SCALE = math.sqrt(float(_D))


@functools.partial(jax.jit, static_argnums=(2,))
def _lookup(x_flat, table, n_chunks):
    mesh = plsc.VectorSubcoreMesh(core_axis_name="c", subcore_axis_name="s")

    @functools.partial(
        pl.kernel,
        out_type=jax.ShapeDtypeStruct((_NW * n_chunks * _C, _D), jnp.float32),
        mesh=mesh,
        scratch_types=[
            pltpu.VMEM((n_chunks, _C), jnp.int32),
            *[pltpu.VMEM((_C, _D), jnp.float32) for _ in range(_NBUF)],
            *[pltpu.SemaphoreType.DMA for _ in range(2 * _NBUF)],
        ],
    )
    def k(x_hbm, tab_hbm, out_hbm, idx_v, *rest):
        bufs = rest[:_NBUF]
        in_sems = rest[_NBUF:2 * _NBUF]
        out_sems = rest[2 * _NBUF:]
        wid = lax.axis_index("s") * _NC + lax.axis_index("c")
        base0 = wid * n_chunks * _C

        pltpu.sync_copy(x_hbm.at[wid], idx_v)

        for b in range(_NBUF):
            pltpu.async_copy(tab_hbm.at[idx_v.at[b]], bufs[b], in_sems[b], priority=1)

        def scale_buf(buf):
            @plsc.parallel_loop(0, _C, unroll=4)
            def _(r):
                for j in range(_D // _L):
                    sl = (r, pl.ds(j * _L, _L))
                    buf[sl] = buf[sl] * _SCALE

        def outer(t, _):
            gg = t * _NBUF
            for b in range(_NBUF):
                g = gg + b
                pltpu.make_async_copy(
                    tab_hbm.at[idx_v.at[g]], bufs[b], in_sems[b]).wait()
                scale_buf(bufs[b])
                pltpu.async_copy(
                    bufs[b], out_hbm.at[pl.ds(base0 + g * _C, _C)],
                    out_sems[b])
                pb = (b - 1) % _NBUF
                pg = g + _NBUF - 1

                @pl.when(jnp.logical_and(g >= 1, pg < n_chunks))
                def _():
                    prow = base0 + (pg - _NBUF) * _C
                    pltpu.make_async_copy(
                        bufs[pb], out_hbm.at[pl.ds(prow, _C)],
                        out_sems[pb]).wait()
                    pltpu.async_copy(
                        tab_hbm.at[idx_v.at[pg]], bufs[pb], in_sems[pb],
                        priority=1)
            return 0

        lax.fori_loop(0, n_chunks // _NBUF, outer, 0)

        for b in range(_NBUF):
            g = n_chunks - _NBUF + b
            pltpu.make_async_copy(
                bufs[b], out_hbm.at[pl.ds(base0 + g * _C, _C)],
                out_sems[b]).wait()

    return k(x_flat, table)


def kernel(x, table):
    b, h = x.shape
    n = b * h
    assert n % (_NW * _C * _NBUF) == 0
    n_chunks = n // (_NW * _C)
    x_r = x.astype(jnp.int32).reshape(_NW, n_chunks, _C)
    out = _lookup(x_r, table, n_chunks)
    return out.reshape(b, h, _D)


# speedup vs baseline: 1.0045x; 1.0045x over previous
import functools
import math

import jax
import jax.numpy as jnp
from jax import lax
from jax.experimental import pallas as pl
from jax.experimental.pallas import tpu as pltpu
from jax.experimental.pallas import tpu_sc as plsc

_D = 128
_L = 16
_NC = 2
_NS = 16
_NW = _NC * _NS
_C = 128
_NBUF = 4
_SCALE = math.sqrt(float(_D))


@functools.partial(jax.jit, static_argnums=(2,))
def _lookup(x_flat, table, n_chunks):
    mesh = plsc.VectorSubcoreMesh(core_axis_name="c", subcore_axis_name="s")

    @functools.partial(
        pl.kernel,
        out_type=jax.ShapeDtypeStruct((_NW * n_chunks * _C, _D), jnp.float32),
        mesh=mesh,
        scratch_types=[
            pltpu.VMEM((n_chunks, _C), jnp.int32),
            *[pltpu.VMEM((_C, _D), jnp.float32) for _ in range(_NBUF)],
            *[pltpu.SemaphoreType.DMA for _ in range(2 * _NBUF)],
        ],
    )
    def k(x_hbm, tab_hbm, out_hbm, idx_v, *rest):
        bufs = rest[:_NBUF]
        in_sems = rest[_NBUF:2 * _NBUF]
        out_sems = rest[2 * _NBUF:]
        wid = lax.axis_index("s") * _NC + lax.axis_index("c")
        base0 = wid * n_chunks * _C

        pltpu.sync_copy(x_hbm.at[wid], idx_v)

        for b in range(_NBUF):
            pltpu.async_copy(tab_hbm.at[idx_v.at[b]], bufs[b], in_sems[b])

        def scale_buf(buf):
            @plsc.parallel_loop(0, _C, unroll=4)
            def _(r):
                for j in range(_D // _L):
                    sl = (r, pl.ds(j * _L, _L))
                    buf[sl] = buf[sl] * _SCALE

        def outer(t, _):
            gg = t * _NBUF
            for b in range(_NBUF):
                g = gg + b
                pltpu.make_async_copy(
                    tab_hbm.at[idx_v.at[g]], bufs[b], in_sems[b]).wait()
                scale_buf(bufs[b])
                pltpu.async_copy(
                    bufs[b], out_hbm.at[pl.ds(base0 + g * _C, _C)],
                    out_sems[b])
                pb = (b - 1) % _NBUF
                pg = g + _NBUF - 1

                @pl.when(jnp.logical_and(g >= 1, pg < n_chunks))
                def _():
                    prow = base0 + (pg - _NBUF) * _C
                    pltpu.make_async_copy(
                        bufs[pb], out_hbm.at[pl.ds(prow, _C)],
                        out_sems[pb]).wait()
                    pltpu.async_copy(
                        tab_hbm.at[idx_v.at[pg]], bufs[pb], in_sems[pb])
            return 0

        lax.fori_loop(0, n_chunks // _NBUF, outer, 0)

        for b in range(_NBUF):
            g = n_chunks - _NBUF + b
            pltpu.make_async_copy(
                bufs[b], out_hbm.at[pl.ds(base0 + g * _C, _C)],
                out_sems[b]).wait()

    return k(x_flat, table)


def kernel(x, table):
    b, h = x.shape
    n = b * h
    assert n % (_NW * _C * _NBUF) == 0
    n_chunks = n // (_NW * _C)
    x_r = x.astype(jnp.int32).reshape(_NW, n_chunks, _C)
    out = _lookup(x_r, table, n_chunks)
    return out.reshape(b, h, _D)
